# baseline (speedup 1.0000x reference)
import jax
import jax.numpy as jnp
from jax.experimental import pallas as pl
from jax.experimental.pallas import tpu as pltpu

BM = 256


def _mm_body(x_ref, m_ref, w_ref, b_ref, o_ref):
    m = m_ref[...]
    xm = x_ref[...] * m
    acc = jnp.dot(xm, w_ref[...], preferred_element_type=jnp.float32)
    o_ref[...] = acc + m * b_ref[...]


def kernel(inputs, mask, W, b):
    B, T, H = inputs.shape
    D = W.shape[1]
    M = B * T
    x = inputs.reshape(M, H)
    mf = mask.reshape(M, 1).astype(jnp.float32)
    b2 = b.reshape(1, D)

    out = pl.pallas_call(
        _mm_body,
        grid=(M // BM,),
        in_specs=[
            pl.BlockSpec((BM, H), lambda i: (i, 0)),
            pl.BlockSpec((BM, 1), lambda i: (i, 0)),
            pl.BlockSpec((H, D), lambda i: (0, 0)),
            pl.BlockSpec((1, D), lambda i: (0, 0)),
        ],
        out_specs=pl.BlockSpec((BM, D), lambda i: (i, 0)),
        out_shape=jax.ShapeDtypeStruct((M, D), jnp.float32),
        compiler_params=pltpu.CompilerParams(
            dimension_semantics=("arbitrary",),
        ),
    )(x, mf, W, b2)
    return out.reshape(B, T, D)
